# baseline (speedup 1.0000x reference)
import functools
import math

import jax
import jax.numpy as jnp
import numpy as np
from jax import lax
from jax.experimental import pallas as pl
from jax.experimental.pallas import tpu as pltpu
from jax.experimental.pallas import tpu_sc as plsc

EMBED_NUM = 1000000
HIDDEN_DIM = 32
LANES = 16
PI = np.float32(math.pi)
HALF = np.float32(EMBED_NUM // 2)
ONE = np.float32(1.0)

NUM_CORES = 2
NUM_SUBCORES = 16
NUM_WORKERS = NUM_CORES * NUM_SUBCORES

SEQ = 50
SAMP_PER_CHUNK = 8
CHUNK = SAMP_PER_CHUNK * SEQ
G_IDX = 100
G_PER_CHUNK = CHUNK // G_IDX
NBUF = 2


def _sc_embed(index_flat, table, *, n_samples):
    samp_per_w = n_samples // NUM_WORKERS
    n_chunks = samp_per_w // SAMP_PER_CHUNK
    assert n_chunks % NBUF == 0
    mesh = plsc.VectorSubcoreMesh(core_axis_name="c", subcore_axis_name="s")

    @functools.partial(
        pl.kernel,
        mesh=mesh,
        out_type=jax.ShapeDtypeStruct((n_samples, SEQ, HIDDEN_DIM),
                                      jnp.float32),
        scratch_types=[
            pltpu.VMEM((NBUF, CHUNK), jnp.float32),
            pltpu.VMEM((NBUF * G_PER_CHUNK, G_IDX), jnp.int32),
            pltpu.VMEM((NBUF, CHUNK, HIDDEN_DIM), jnp.float32),
            pltpu.SemaphoreType.DMA,
            pltpu.SemaphoreType.DMA,
            pltpu.SemaphoreType.DMA,
            pltpu.SemaphoreType.DMA,
        ],
        compiler_params=pltpu.CompilerParams(use_tc_tiling_on_sc=False),
    )
    def body(ang_hbm, table_hbm, out_hbm, ang_v, idx_v, rows_v, sg0, sg1,
             ss0, ss1):
        wid = lax.axis_index("s") * NUM_CORES + lax.axis_index("c")
        samp_base = wid * samp_per_w
        sem_g = (sg0, sg1)
        sem_st = (ss0, ss1)

        def gather_copies(b):
            return [
                pltpu.make_async_copy(
                    table_hbm.at[idx_v.at[b * G_PER_CHUNK + j]],
                    rows_v.at[b, pl.ds(j * G_IDX, G_IDX)],
                    sem_g[b],
                )
                for j in range(G_PER_CHUNK)
            ]

        def store_copies(b, ci):
            s0 = samp_base + ci * SAMP_PER_CHUNK
            return [
                pltpu.make_async_copy(
                    rows_v.at[b, pl.ds(s * SEQ, SEQ)],
                    out_hbm.at[s0 + s],
                    sem_st[b],
                )
                for s in range(SAMP_PER_CHUNK)
            ]

        def compute_idx(b, ci):
            off = (samp_base + ci * SAMP_PER_CHUNK) * SEQ
            pltpu.sync_copy(ang_hbm.at[pl.ds(off, CHUNK)], ang_v.at[b])
            for j in range(G_PER_CHUNK):
                starts = [i * LANES for i in range(G_IDX // LANES)]
                starts.append(G_IDX - LANES)
                for s in starts:
                    x = ang_v[b, pl.ds(j * G_IDX + s, LANES)]
                    y = (x / PI + ONE) * HALF
                    ii = y.astype(jnp.int32)
                    ii = jnp.minimum(jnp.maximum(ii, 0), EMBED_NUM - 1)
                    idx_v[b * G_PER_CHUNK + j, pl.ds(s, LANES)] = ii

        def outer(gi, _):
            for b in range(NBUF):
                ci = gi * NBUF + b
                pb = 1 - b
                compute_idx(b, ci)
                @pl.when(ci >= NBUF)
                def _():
                    for c in store_copies(b, ci - NBUF):
                        c.wait()
                for c in gather_copies(b):
                    c.start()
                @pl.when(ci >= 1)
                def _():
                    for c in gather_copies(pb):
                        c.wait()
                    for c in store_copies(pb, ci - 1):
                        c.start()
            return 0

        lax.fori_loop(0, n_chunks // NBUF, outer, 0)
        last = n_chunks - 1
        lb = last % NBUF
        for c in gather_copies(lb):
            c.wait()
        for c in store_copies(lb, last):
            c.start()
        for c in store_copies(1 - lb, last - 1):
            c.wait()
        for c in store_copies(lb, last):
            c.wait()

    return body(index_flat, table)


K0_ROWS = 64
K0_NBUF = 2


def _sc_detile(table):
    total_chunks = EMBED_NUM // K0_ROWS
    per_w = (total_chunks // NUM_WORKERS // K0_NBUF) * K0_NBUF
    tail = total_chunks - per_w * NUM_WORKERS
    vals = K0_ROWS * HIDDEN_DIM
    mesh = plsc.VectorSubcoreMesh(core_axis_name="c", subcore_axis_name="s")

    @functools.partial(
        pl.kernel,
        mesh=mesh,
        out_type=jax.ShapeDtypeStruct((EMBED_NUM * HIDDEN_DIM,), jnp.float32),
        scratch_types=[
            pltpu.VMEM((K0_ROWS, HIDDEN_DIM), jnp.float32),
            pltpu.VMEM((K0_ROWS, HIDDEN_DIM), jnp.float32),
            pltpu.VMEM((vals,), jnp.float32),
            pltpu.VMEM((vals,), jnp.float32),
            pltpu.SemaphoreType.DMA,
            pltpu.SemaphoreType.DMA,
            pltpu.SemaphoreType.DMA,
            pltpu.SemaphoreType.DMA,
        ],
    )
    def body(table_hbm, out_hbm, td0, td1, fl0, fl1, sl0, sl1, ss0, ss1):
        wid = lax.axis_index("s") * NUM_CORES + lax.axis_index("c")
        td = (td0, td1)
        fl = (fl0, fl1)
        sem_l = (sl0, sl1)
        sem_st = (ss0, ss1)

        def chunk_id(j):
            return wid + j * NUM_WORKERS

        def load_copy(b, ci):
            return pltpu.make_async_copy(
                table_hbm.at[pl.ds(ci * K0_ROWS, K0_ROWS)], td[b], sem_l[b])

        def store_copy(b, ci):
            return pltpu.make_async_copy(
                fl[b], out_hbm.at[pl.ds(ci * vals, vals)], sem_st[b])

        def repack(b):
            for r in range(K0_ROWS):
                for h in range(HIDDEN_DIM // LANES):
                    fl[b][pl.ds(r * HIDDEN_DIM + h * LANES, LANES)] = (
                        td[b][r, pl.ds(h * LANES, LANES)])

        load_copy(0, chunk_id(0)).start()

        def outer(gi, _):
            for b in range(K0_NBUF):
                j = gi * K0_NBUF + b
                load_copy(b, chunk_id(j)).wait()
                @pl.when(j + 1 < per_w)
                def _():
                    load_copy(1 - b, chunk_id(j + 1)).start()
                @pl.when(j >= K0_NBUF)
                def _():
                    store_copy(b, chunk_id(j - K0_NBUF)).wait()
                repack(b)
                store_copy(b, chunk_id(j)).start()
            return 0

        lax.fori_loop(0, per_w // K0_NBUF, outer, 0)
        store_copy(0, chunk_id(per_w - 2)).wait()
        store_copy(1, chunk_id(per_w - 1)).wait()
        @pl.when(wid < tail)
        def _():
            ci = per_w * NUM_WORKERS + wid
            load_copy(0, ci).start()
            load_copy(0, ci).wait()
            repack(0)
            store_copy(0, ci).start()
            store_copy(0, ci).wait()

    return body(table)


def kernel(index, table):
    n_samples = index.shape[0]
    flat = index.reshape(n_samples * SEQ)
    table_1d = _sc_detile(table)
    table_c = table_1d.reshape(EMBED_NUM, HIDDEN_DIM)
    return _sc_embed(flat, table_c, n_samples=n_samples)


# speedup vs baseline: 1.3087x; 1.3087x over previous
import functools
import math

import jax
import jax.numpy as jnp
import numpy as np
from jax import lax
from jax.experimental import pallas as pl
from jax.experimental.pallas import tpu as pltpu
from jax.experimental.pallas import tpu_sc as plsc

EMBED_NUM = 1000000
HIDDEN_DIM = 32
LANES = 16
PI = np.float32(math.pi)
HALF = np.float32(EMBED_NUM // 2)
ONE = np.float32(1.0)

NUM_CORES = 2
NUM_SUBCORES = 16
NUM_WORKERS = NUM_CORES * NUM_SUBCORES

SEQ = 50
SAMP_PER_CHUNK = 8
CHUNK = SAMP_PER_CHUNK * SEQ
G_IDX = 100
G_PER_CHUNK = CHUNK // G_IDX
NBUF = 2


def _sc_embed(index_flat, table, *, n_samples):
    samp_per_w = n_samples // NUM_WORKERS
    n_chunks = samp_per_w // SAMP_PER_CHUNK
    assert n_chunks % NBUF == 0
    mesh = plsc.VectorSubcoreMesh(core_axis_name="c", subcore_axis_name="s")

    @functools.partial(
        pl.kernel,
        mesh=mesh,
        out_type=jax.ShapeDtypeStruct((n_samples, SEQ, HIDDEN_DIM),
                                      jnp.float32),
        scratch_types=[
            pltpu.VMEM((NBUF, CHUNK), jnp.float32),
            pltpu.VMEM((NBUF * G_PER_CHUNK, G_IDX), jnp.int32),
            pltpu.VMEM((NBUF, CHUNK, HIDDEN_DIM), jnp.float32),
            pltpu.SemaphoreType.DMA,
            pltpu.SemaphoreType.DMA,
            pltpu.SemaphoreType.DMA,
            pltpu.SemaphoreType.DMA,
        ],
        compiler_params=pltpu.CompilerParams(use_tc_tiling_on_sc=False),
    )
    def body(ang_hbm, table_hbm, out_hbm, ang_v, idx_v, rows_v, sg0, sg1,
             ss0, ss1):
        wid = lax.axis_index("s") * NUM_CORES + lax.axis_index("c")
        samp_base = wid * samp_per_w
        sem_g = (sg0, sg1)
        sem_st = (ss0, ss1)

        def gather_copies(b):
            return [
                pltpu.make_async_copy(
                    table_hbm.at[idx_v.at[b * G_PER_CHUNK + j]],
                    rows_v.at[b, pl.ds(j * G_IDX, G_IDX)],
                    sem_g[b],
                )
                for j in range(G_PER_CHUNK)
            ]

        def store_copies(b, ci):
            s0 = samp_base + ci * SAMP_PER_CHUNK
            return [
                pltpu.make_async_copy(
                    rows_v.at[b, pl.ds(s * SEQ, SEQ)],
                    out_hbm.at[s0 + s],
                    sem_st[b],
                )
                for s in range(SAMP_PER_CHUNK)
            ]

        def compute_idx(b, ci):
            off = (samp_base + ci * SAMP_PER_CHUNK) * SEQ
            pltpu.sync_copy(ang_hbm.at[pl.ds(off, CHUNK)], ang_v.at[b])
            for j in range(G_PER_CHUNK):
                starts = [i * LANES for i in range(G_IDX // LANES)]
                starts.append(G_IDX - LANES)
                for s in starts:
                    x = ang_v[b, pl.ds(j * G_IDX + s, LANES)]
                    y = (x / PI + ONE) * HALF
                    ii = y.astype(jnp.int32)
                    ii = jnp.minimum(jnp.maximum(ii, 0), EMBED_NUM - 1)
                    idx_v[b * G_PER_CHUNK + j, pl.ds(s, LANES)] = ii

        def outer(gi, _):
            for b in range(NBUF):
                ci = gi * NBUF + b
                pb = 1 - b
                compute_idx(b, ci)
                @pl.when(ci >= NBUF)
                def _():
                    for c in store_copies(b, ci - NBUF):
                        c.wait()
                for c in gather_copies(b):
                    c.start()
                @pl.when(ci >= 1)
                def _():
                    for c in gather_copies(pb):
                        c.wait()
                    for c in store_copies(pb, ci - 1):
                        c.start()
            return 0

        lax.fori_loop(0, n_chunks // NBUF, outer, 0)
        last = n_chunks - 1
        lb = last % NBUF
        for c in gather_copies(lb):
            c.wait()
        for c in store_copies(lb, last):
            c.start()
        for c in store_copies(1 - lb, last - 1):
            c.wait()
        for c in store_copies(lb, last):
            c.wait()

    return body(index_flat, table)


def kernel(index, table):
    n_samples = index.shape[0]
    flat = index.reshape(n_samples * SEQ)
    return _sc_embed(flat, table, n_samples=n_samples)


# speedup vs baseline: 1.3157x; 1.0054x over previous
import functools
import math

import jax
import jax.numpy as jnp
import numpy as np
from jax import lax
from jax.experimental import pallas as pl
from jax.experimental.pallas import tpu as pltpu
from jax.experimental.pallas import tpu_sc as plsc

EMBED_NUM = 1000000
HIDDEN_DIM = 32
LANES = 16
PI = np.float32(math.pi)
HALF = np.float32(EMBED_NUM // 2)
ONE = np.float32(1.0)

NUM_CORES = 2
NUM_SUBCORES = 16
NUM_WORKERS = NUM_CORES * NUM_SUBCORES

SEQ = 50
SAMP_PER_CHUNK = 16
CHUNK = SAMP_PER_CHUNK * SEQ
G_IDX = 100
G_PER_CHUNK = CHUNK // G_IDX
NBUF = 2


def _sc_embed(index_flat, table, *, n_samples):
    samp_per_w = n_samples // NUM_WORKERS
    n_chunks = samp_per_w // SAMP_PER_CHUNK
    assert n_chunks % NBUF == 0
    mesh = plsc.VectorSubcoreMesh(core_axis_name="c", subcore_axis_name="s")

    @functools.partial(
        pl.kernel,
        mesh=mesh,
        out_type=jax.ShapeDtypeStruct((n_samples, SEQ, HIDDEN_DIM),
                                      jnp.float32),
        scratch_types=[
            pltpu.VMEM((NBUF, CHUNK), jnp.float32),
            pltpu.VMEM((NBUF * G_PER_CHUNK, G_IDX), jnp.int32),
            pltpu.VMEM((NBUF, CHUNK, HIDDEN_DIM), jnp.float32),
            pltpu.SemaphoreType.DMA,
            pltpu.SemaphoreType.DMA,
            pltpu.SemaphoreType.DMA,
            pltpu.SemaphoreType.DMA,
        ],
        compiler_params=pltpu.CompilerParams(use_tc_tiling_on_sc=False),
    )
    def body(ang_hbm, table_hbm, out_hbm, ang_v, idx_v, rows_v, sg0, sg1,
             ss0, ss1):
        wid = lax.axis_index("s") * NUM_CORES + lax.axis_index("c")
        samp_base = wid * samp_per_w
        sem_g = (sg0, sg1)
        sem_st = (ss0, ss1)

        def gather_copies(b):
            return [
                pltpu.make_async_copy(
                    table_hbm.at[idx_v.at[b * G_PER_CHUNK + j]],
                    rows_v.at[b, pl.ds(j * G_IDX, G_IDX)],
                    sem_g[b],
                )
                for j in range(G_PER_CHUNK)
            ]

        def store_copies(b, ci):
            s0 = samp_base + ci * SAMP_PER_CHUNK
            return [
                pltpu.make_async_copy(
                    rows_v.at[b, pl.ds(s * SEQ, SEQ)],
                    out_hbm.at[s0 + s],
                    sem_st[b],
                )
                for s in range(SAMP_PER_CHUNK)
            ]

        def compute_idx(b, ci):
            off = (samp_base + ci * SAMP_PER_CHUNK) * SEQ
            pltpu.sync_copy(ang_hbm.at[pl.ds(off, CHUNK)], ang_v.at[b])
            for j in range(G_PER_CHUNK):
                starts = [i * LANES for i in range(G_IDX // LANES)]
                starts.append(G_IDX - LANES)
                for s in starts:
                    x = ang_v[b, pl.ds(j * G_IDX + s, LANES)]
                    y = (x / PI + ONE) * HALF
                    ii = y.astype(jnp.int32)
                    ii = jnp.minimum(jnp.maximum(ii, 0), EMBED_NUM - 1)
                    idx_v[b * G_PER_CHUNK + j, pl.ds(s, LANES)] = ii

        def outer(gi, _):
            for b in range(NBUF):
                ci = gi * NBUF + b
                pb = 1 - b
                compute_idx(b, ci)
                @pl.when(ci >= NBUF)
                def _():
                    for c in store_copies(b, ci - NBUF):
                        c.wait()
                for c in gather_copies(b):
                    c.start()
                @pl.when(ci >= 1)
                def _():
                    for c in gather_copies(pb):
                        c.wait()
                    for c in store_copies(pb, ci - 1):
                        c.start()
            return 0

        lax.fori_loop(0, n_chunks // NBUF, outer, 0)
        last = n_chunks - 1
        lb = last % NBUF
        for c in gather_copies(lb):
            c.wait()
        for c in store_copies(lb, last):
            c.start()
        for c in store_copies(1 - lb, last - 1):
            c.wait()
        for c in store_copies(lb, last):
            c.wait()

    return body(index_flat, table)


def kernel(index, table):
    n_samples = index.shape[0]
    flat = index.reshape(n_samples * SEQ)
    return _sc_embed(flat, table, n_samples=n_samples)
